# initial kernel scaffold (unmeasured)
import jax
import jax.numpy as jnp
from jax import lax
from jax.experimental import pallas as pl
from jax.experimental.pallas import tpu as pltpu


def kernel(
    x,
):
    def body(*refs):
        pass

    out_shape = jax.ShapeDtypeStruct(..., jnp.float32)
    return pl.pallas_call(body, out_shape=out_shape)(...)


# baseline (device time: 807496 ns/iter reference)
import jax
import jax.numpy as jnp
from jax import lax
from jax.experimental import pallas as pl
from jax.experimental.pallas import tpu as pltpu

N_CHUNKS = 16


def kernel(x):
    _, m_total, n_total = x.shape
    n_half = n_total // 2
    m_c = m_total // N_CHUNKS

    def body(x_ref, out_ref, comm_ref, send_sems, recv_sems):
        step = pl.program_id(0)
        my_x = lax.axis_index("x")
        my_y = lax.axis_index("y")
        my_z = lax.axis_index("z")
        partner = (my_x, my_y, 1 - my_z)

        @pl.when(step == 0)
        def _():
            barrier_sem = pltpu.get_barrier_semaphore()
            pl.semaphore_signal(
                barrier_sem,
                inc=1,
                device_id=partner,
                device_id_type=pl.DeviceIdType.MESH,
            )
            pl.semaphore_wait(barrier_sem, 1)

        slot = step % 2
        send_off = (1 - my_z) * n_half
        keep_off = my_z * n_half

        rdma = pltpu.make_async_remote_copy(
            src_ref=x_ref.at[0, :, pl.ds(send_off, n_half)],
            dst_ref=comm_ref.at[slot],
            send_sem=send_sems.at[slot],
            recv_sem=recv_sems.at[slot],
            device_id=partner,
            device_id_type=pl.DeviceIdType.MESH,
        )
        rdma.start()
        rdma.wait()

        out_ref[:, :] = x_ref[0, :, pl.ds(keep_off, n_half)] + comm_ref[slot]

    return pl.pallas_call(
        body,
        grid=(N_CHUNKS,),
        out_shape=jax.ShapeDtypeStruct((m_total, n_half), jnp.float32),
        in_specs=[
            pl.BlockSpec(
                (1, m_c, n_total), lambda i: (0, i, 0), memory_space=pltpu.VMEM
            )
        ],
        out_specs=pl.BlockSpec(
            (m_c, n_half), lambda i: (i, 0), memory_space=pltpu.VMEM
        ),
        scratch_shapes=[
            pltpu.VMEM((2, m_c, n_half), jnp.float32),
            pltpu.SemaphoreType.DMA((2,)),
            pltpu.SemaphoreType.DMA((2,)),
        ],
        compiler_params=pltpu.CompilerParams(
            collective_id=0, dimension_semantics=("arbitrary",)
        ),
    )(x)


# device time: 772095 ns/iter; 1.0459x vs baseline; 1.0459x over previous
import jax
import jax.numpy as jnp
from jax import lax
from jax.experimental import pallas as pl
from jax.experimental.pallas import tpu as pltpu

N_CHUNKS = 32
N_SLOTS = 4


def kernel(x):
    _, m_total, n_total = x.shape
    n_half = n_total // 2
    m_c = m_total // N_CHUNKS

    def body(
        z_ref,
        x_send_ref,
        x_keep_ref,
        out_ref,
        send_buf,
        comm_ref,
        send_sems,
        recv_sems,
    ):
        t = pl.program_id(0)
        my_x = lax.axis_index("x")
        my_y = lax.axis_index("y")
        my_z = lax.axis_index("z")
        partner = (my_x, my_y, 1 - my_z)

        def make_rdma(slot):
            return pltpu.make_async_remote_copy(
                src_ref=send_buf.at[slot],
                dst_ref=comm_ref.at[slot],
                send_sem=send_sems.at[slot],
                recv_sem=recv_sems.at[slot],
                device_id=partner,
                device_id_type=pl.DeviceIdType.MESH,
            )

        @pl.when(t == 0)
        def _():
            barrier_sem = pltpu.get_barrier_semaphore()
            pl.semaphore_signal(
                barrier_sem,
                inc=1,
                device_id=partner,
                device_id_type=pl.DeviceIdType.MESH,
            )
            pl.semaphore_wait(barrier_sem, 1)

        @pl.when(t < N_CHUNKS)
        def _():
            slot = t % N_SLOTS

            @pl.when(t >= N_SLOTS)
            def _():
                make_rdma(slot).wait_send()

            send_buf[slot] = x_send_ref[0]
            make_rdma(slot).start()

        @pl.when(t >= 1)
        def _():
            j = t - 1
            slot = j % N_SLOTS
            make_rdma(slot).wait_recv()
            out_ref[:, :] = x_keep_ref[0] + comm_ref[slot]

        @pl.when(t == N_CHUNKS)
        def _():
            for s in range(N_SLOTS):
                make_rdma(s).wait_send()

    grid_spec = pltpu.PrefetchScalarGridSpec(
        num_scalar_prefetch=1,
        grid=(N_CHUNKS + 1,),
        in_specs=[
            pl.BlockSpec(
                (1, m_c, n_half),
                lambda t, zr: (0, jnp.minimum(t, N_CHUNKS - 1), 1 - zr[0]),
            ),
            pl.BlockSpec(
                (1, m_c, n_half),
                lambda t, zr: (0, jnp.maximum(t - 1, 0), zr[0]),
            ),
        ],
        out_specs=pl.BlockSpec(
            (m_c, n_half), lambda t, zr: (jnp.maximum(t - 1, 0), 0)
        ),
        scratch_shapes=[
            pltpu.VMEM((N_SLOTS, m_c, n_half), jnp.float32),
            pltpu.VMEM((N_SLOTS, m_c, n_half), jnp.float32),
            pltpu.SemaphoreType.DMA((N_SLOTS,)),
            pltpu.SemaphoreType.DMA((N_SLOTS,)),
        ],
    )

    z_idx = jnp.reshape(lax.axis_index("z"), (1,)).astype(jnp.int32)
    return pl.pallas_call(
        body,
        grid_spec=grid_spec,
        out_shape=jax.ShapeDtypeStruct((m_total, n_half), jnp.float32),
        compiler_params=pltpu.CompilerParams(
            collective_id=0, dimension_semantics=("arbitrary",)
        ),
    )(z_idx, x, x)


# device time: 771978 ns/iter; 1.0460x vs baseline; 1.0002x over previous
import jax
import jax.numpy as jnp
from jax import lax
from jax.experimental import pallas as pl
from jax.experimental.pallas import tpu as pltpu

N_CHUNKS = 32
LAG = 2
N_RECV_SLOTS = 6
N_SEND_SLOTS = 3


def kernel(x):
    _, m_total, n_total = x.shape
    n_half = n_total // 2
    m_c = m_total // N_CHUNKS

    def body(
        z_ref,
        x_send_ref,
        x_keep_ref,
        out_ref,
        send_buf,
        comm_ref,
        send_sems,
        recv_sems,
    ):
        t = pl.program_id(0)
        my_x = lax.axis_index("x")
        my_y = lax.axis_index("y")
        my_z = lax.axis_index("z")
        partner = (my_x, my_y, 1 - my_z)

        def make_rdma(chunk):
            return pltpu.make_async_remote_copy(
                src_ref=send_buf.at[chunk % N_SEND_SLOTS],
                dst_ref=comm_ref.at[chunk % N_RECV_SLOTS],
                send_sem=send_sems.at[chunk % N_RECV_SLOTS],
                recv_sem=recv_sems.at[chunk % N_RECV_SLOTS],
                device_id=partner,
                device_id_type=pl.DeviceIdType.MESH,
            )

        @pl.when(t == 0)
        def _():
            barrier_sem = pltpu.get_barrier_semaphore()
            pl.semaphore_signal(
                barrier_sem,
                inc=1,
                device_id=partner,
                device_id_type=pl.DeviceIdType.MESH,
            )
            pl.semaphore_wait(barrier_sem, 1)

        @pl.when(t < N_CHUNKS)
        def _():
            @pl.when(t >= N_SEND_SLOTS)
            def _():
                make_rdma(t - N_SEND_SLOTS).wait_send()

            send_buf[t % N_SEND_SLOTS] = x_send_ref[0]
            make_rdma(t).start()

        @pl.when(t >= LAG)
        def _():
            j = t - LAG
            make_rdma(j).wait_recv()
            out_ref[:, :] = x_keep_ref[0] + comm_ref[j % N_RECV_SLOTS]

        @pl.when(t == N_CHUNKS + LAG - 1)
        def _():
            for c in range(N_CHUNKS - N_SEND_SLOTS, N_CHUNKS):
                make_rdma(c).wait_send()

    grid_spec = pltpu.PrefetchScalarGridSpec(
        num_scalar_prefetch=1,
        grid=(N_CHUNKS + LAG,),
        in_specs=[
            pl.BlockSpec(
                (1, m_c, n_half),
                lambda t, zr: (0, jnp.minimum(t, N_CHUNKS - 1), 1 - zr[0]),
            ),
            pl.BlockSpec(
                (1, m_c, n_half),
                lambda t, zr: (0, jnp.maximum(t - LAG, 0), zr[0]),
            ),
        ],
        out_specs=pl.BlockSpec(
            (m_c, n_half), lambda t, zr: (jnp.maximum(t - LAG, 0), 0)
        ),
        scratch_shapes=[
            pltpu.VMEM((N_SEND_SLOTS, m_c, n_half), jnp.float32),
            pltpu.VMEM((N_RECV_SLOTS, m_c, n_half), jnp.float32),
            pltpu.SemaphoreType.DMA((N_RECV_SLOTS,)),
            pltpu.SemaphoreType.DMA((N_RECV_SLOTS,)),
        ],
    )

    z_idx = jnp.reshape(lax.axis_index("z"), (1,)).astype(jnp.int32)
    return pl.pallas_call(
        body,
        grid_spec=grid_spec,
        out_shape=jax.ShapeDtypeStruct((m_total, n_half), jnp.float32),
        compiler_params=pltpu.CompilerParams(
            collective_id=0, dimension_semantics=("arbitrary",)
        ),
    )(z_idx, x, x)


# device time: 506913 ns/iter; 1.5930x vs baseline; 1.5229x over previous
import jax
import jax.numpy as jnp
from jax import lax
from jax.experimental import pallas as pl
from jax.experimental.pallas import tpu as pltpu

N_CHUNKS = 8


def _ring_ids():
    my_x = lax.axis_index("x")
    my_y = lax.axis_index("y")
    my_z = lax.axis_index("z")
    p = my_x * 2 + jnp.where(my_x == 0, my_y, 1 - my_y)
    cond = my_x == my_y
    cw = (
        jnp.where(cond, my_x, 1 - my_x),
        jnp.where(cond, 1 - my_y, my_y),
        my_z,
    )
    ccw = (
        jnp.where(cond, 1 - my_x, my_x),
        jnp.where(cond, my_y, 1 - my_y),
        my_z,
    )
    zpartner = (my_x, my_y, 1 - my_z)
    return p, cw, ccw, zpartner, my_z


def _zreduce_kernel(x):
    _, m_total, n_total = x.shape
    n_half = n_total // 2
    m_q = m_total // 4
    m_c = m_q // N_CHUNKS

    def body(x_ref, oq_ref, zrecv_ref, zs_sems, zr_sems, la, lb, lo, va, vb, vs):
        p, _cw, _ccw, zpartner, my_z = _ring_ids()
        row0 = p * m_q
        send_off = (1 - my_z) * n_half
        keep_off = my_z * n_half

        barrier_sem = pltpu.get_barrier_semaphore()
        pl.semaphore_signal(
            barrier_sem,
            inc=1,
            device_id=zpartner,
            device_id_type=pl.DeviceIdType.MESH,
        )
        pl.semaphore_wait(barrier_sem, 1)

        def chunk_rdma(c):
            return pltpu.make_async_remote_copy(
                src_ref=x_ref.at[0, pl.ds(row0 + c * m_c, m_c), pl.ds(send_off, n_half)],
                dst_ref=zrecv_ref.at[pl.ds(c * m_c, m_c), :],
                send_sem=zs_sems.at[c],
                recv_sem=zr_sems.at[c],
                device_id=zpartner,
                device_id_type=pl.DeviceIdType.MESH,
            )

        for c in range(N_CHUNKS):
            chunk_rdma(c).start()

        for c in range(N_CHUNKS):
            s = c % 2
            if c >= 2:
                pltpu.make_async_copy(
                    vs.at[s], oq_ref.at[pl.ds((c - 2) * m_c, m_c), :], lo.at[s]
                ).wait()
            chunk_rdma(c).wait_recv()
            pltpu.make_async_copy(
                x_ref.at[0, pl.ds(row0 + c * m_c, m_c), pl.ds(keep_off, n_half)],
                va.at[s],
                la.at[s],
            ).start()
            pltpu.make_async_copy(
                zrecv_ref.at[pl.ds(c * m_c, m_c), :], vb.at[s], lb.at[s]
            ).start()
            pltpu.make_async_copy(
                x_ref.at[0, pl.ds(row0 + c * m_c, m_c), pl.ds(keep_off, n_half)],
                va.at[s],
                la.at[s],
            ).wait()
            pltpu.make_async_copy(
                zrecv_ref.at[pl.ds(c * m_c, m_c), :], vb.at[s], lb.at[s]
            ).wait()
            vs[s] = va[s] + vb[s]
            pltpu.make_async_copy(
                vs.at[s], oq_ref.at[pl.ds(c * m_c, m_c), :], lo.at[s]
            ).start()

        for c in range(max(N_CHUNKS - 2, 0), N_CHUNKS):
            pltpu.make_async_copy(
                vs.at[c % 2], oq_ref.at[pl.ds(c * m_c, m_c), :], lo.at[c % 2]
            ).wait()
        for c in range(N_CHUNKS):
            chunk_rdma(c).wait_send()

    oq, _ = pl.pallas_call(
        body,
        out_shape=(
            jax.ShapeDtypeStruct((m_q, n_half), jnp.float32),
            jax.ShapeDtypeStruct((m_q, n_half), jnp.float32),
        ),
        in_specs=[pl.BlockSpec(memory_space=pl.ANY)],
        out_specs=(
            pl.BlockSpec(memory_space=pl.ANY),
            pl.BlockSpec(memory_space=pl.ANY),
        ),
        scratch_shapes=[
            pltpu.SemaphoreType.DMA((N_CHUNKS,)),
            pltpu.SemaphoreType.DMA((N_CHUNKS,)),
            pltpu.SemaphoreType.DMA((2,)),
            pltpu.SemaphoreType.DMA((2,)),
            pltpu.SemaphoreType.DMA((2,)),
            pltpu.VMEM((2, m_c, n_half), jnp.float32),
            pltpu.VMEM((2, m_c, n_half), jnp.float32),
            pltpu.VMEM((2, m_c, n_half), jnp.float32),
        ],
        compiler_params=pltpu.CompilerParams(collective_id=0),
    )(x)
    return oq


def _xy_allgather_kernel(oq, m_total):
    m_q, n_half = oq.shape
    m_h = m_q // 2

    def body(oq_ref, out_ref, s_sems, r_sems, lcopy):
        p, cw, ccw, _zp, _z = _ring_ids()
        left_q = lax.rem(p + 3, 4)
        right_q = lax.rem(p + 1, 4)

        barrier_sem = pltpu.get_barrier_semaphore()
        for nbr in (cw, ccw):
            pl.semaphore_signal(
                barrier_sem,
                inc=1,
                device_id=nbr,
                device_id_type=pl.DeviceIdType.MESH,
            )
        pl.semaphore_wait(barrier_sem, 2)

        def full_rdma(target, r_slot, s_slot):
            return pltpu.make_async_remote_copy(
                src_ref=oq_ref,
                dst_ref=out_ref.at[pl.ds(p * m_q, m_q), :],
                send_sem=s_sems.at[s_slot],
                recv_sem=r_sems.at[r_slot],
                device_id=target,
                device_id_type=pl.DeviceIdType.MESH,
            )

        def relay_rdma(q, half, target, r_slot, s_slot):
            rows = pl.ds(q * m_q + half * m_h, m_h)
            return pltpu.make_async_remote_copy(
                src_ref=out_ref.at[rows, :],
                dst_ref=out_ref.at[rows, :],
                send_sem=s_sems.at[s_slot],
                recv_sem=r_sems.at[r_slot],
                device_id=target,
                device_id_type=pl.DeviceIdType.MESH,
            )

        pltpu.make_async_copy(
            oq_ref, out_ref.at[pl.ds(p * m_q, m_q), :], lcopy
        ).start()
        full_rdma(cw, 0, 0).start()
        full_rdma(ccw, 1, 1).start()

        full_rdma(cw, 0, 0).wait_recv()
        relay_rdma(left_q, 0, cw, 2, 2).start()

        full_rdma(ccw, 1, 1).wait_recv()
        relay_rdma(right_q, 1, ccw, 3, 3).start()

        relay_rdma(left_q, 0, cw, 2, 2).wait_recv()
        relay_rdma(right_q, 1, ccw, 3, 3).wait_recv()

        pltpu.make_async_copy(
            oq_ref, out_ref.at[pl.ds(p * m_q, m_q), :], lcopy
        ).wait()
        full_rdma(cw, 0, 0).wait_send()
        full_rdma(ccw, 1, 1).wait_send()
        relay_rdma(left_q, 0, cw, 2, 2).wait_send()
        relay_rdma(right_q, 1, ccw, 3, 3).wait_send()

    return pl.pallas_call(
        body,
        out_shape=jax.ShapeDtypeStruct((m_total, n_half), jnp.float32),
        in_specs=[pl.BlockSpec(memory_space=pl.ANY)],
        out_specs=pl.BlockSpec(memory_space=pl.ANY),
        scratch_shapes=[
            pltpu.SemaphoreType.DMA((4,)),
            pltpu.SemaphoreType.DMA((4,)),
            pltpu.SemaphoreType.DMA,
        ],
        compiler_params=pltpu.CompilerParams(collective_id=1),
    )(oq)


def kernel(x):
    _, m_total, _ = x.shape
    oq = _zreduce_kernel(x)
    return _xy_allgather_kernel(oq, m_total)


# device time: 368404 ns/iter; 2.1919x vs baseline; 1.3760x over previous
import jax
import jax.numpy as jnp
from jax import lax
from jax.experimental import pallas as pl
from jax.experimental.pallas import tpu as pltpu

N_CHUNKS = 8


def kernel(x):
    _, m_total, n_total = x.shape
    n_half = n_total // 2
    m_q = m_total // 4
    m_c = m_q // N_CHUNKS
    m_h = m_c // 2

    def body(
        x_ref,
        out_ref,
        zrecv_ref,
        zs_sems,
        zr_sems,
        sf_cw,
        sf_ccw,
        rf_left,
        rf_right,
        sh_cw,
        sh_ccw,
        rh_left,
        rh_right,
        la,
        lb,
        lo,
        va,
        vb,
        vs,
    ):
        my_x = lax.axis_index("x")
        my_y = lax.axis_index("y")
        my_z = lax.axis_index("z")
        p = my_x * 2 + jnp.where(my_x == 0, my_y, 1 - my_y)
        cond = my_x == my_y
        cw = (jnp.where(cond, my_x, 1 - my_x), jnp.where(cond, 1 - my_y, my_y), my_z)
        ccw = (jnp.where(cond, 1 - my_x, my_x), jnp.where(cond, my_y, 1 - my_y), my_z)
        zpartner = (my_x, my_y, 1 - my_z)

        row0 = p * m_q
        left_q = lax.rem(p + 3, 4)
        right_q = lax.rem(p + 1, 4)
        diag_q = lax.rem(p + 2, 4)
        send_off = (1 - my_z) * n_half
        keep_off = my_z * n_half

        barrier_sem = pltpu.get_barrier_semaphore()
        for nbr in (cw, ccw, zpartner):
            pl.semaphore_signal(
                barrier_sem,
                inc=1,
                device_id=nbr,
                device_id_type=pl.DeviceIdType.MESH,
            )
        pl.semaphore_wait(barrier_sem, 3)

        def z_rdma(c):
            return pltpu.make_async_remote_copy(
                src_ref=x_ref.at[0, pl.ds(row0 + c * m_c, m_c), pl.ds(send_off, n_half)],
                dst_ref=zrecv_ref.at[pl.ds(c * m_c, m_c), :],
                send_sem=zs_sems.at[c],
                recv_sem=zr_sems.at[c],
                device_id=zpartner,
                device_id_type=pl.DeviceIdType.MESH,
            )

        def bcast(c, s, target, ssems, rsems):
            return pltpu.make_async_remote_copy(
                src_ref=vs.at[s],
                dst_ref=out_ref.at[pl.ds(row0 + c * m_c, m_c), :],
                send_sem=ssems.at[c],
                recv_sem=rsems.at[c],
                device_id=target,
                device_id_type=pl.DeviceIdType.MESH,
            )

        def full_wait(c, q, rsems):
            return pltpu.make_async_remote_copy(
                src_ref=vs.at[0],
                dst_ref=out_ref.at[pl.ds(q * m_q + c * m_c, m_c), :],
                send_sem=sf_cw.at[c],
                recv_sem=rsems.at[c],
                device_id=cw,
                device_id_type=pl.DeviceIdType.MESH,
            )

        def relay(c, q, half, target, ssems, rsems):
            rows = pl.ds(q * m_q + c * m_c + half * m_h, m_h)
            return pltpu.make_async_remote_copy(
                src_ref=out_ref.at[rows, :],
                dst_ref=out_ref.at[rows, :],
                send_sem=ssems.at[c],
                recv_sem=rsems.at[c],
                device_id=target,
                device_id_type=pl.DeviceIdType.MESH,
            )

        for c in range(N_CHUNKS):
            z_rdma(c).start()

        for c in range(N_CHUNKS):
            s = c % 2
            if c >= 2:
                pltpu.make_async_copy(
                    vs.at[s], out_ref.at[pl.ds(row0 + (c - 2) * m_c, m_c), :], lo.at[s]
                ).wait()
                bcast(c - 2, s, cw, sf_cw, rf_left).wait_send()
                bcast(c - 2, s, ccw, sf_ccw, rf_right).wait_send()

            z_rdma(c).wait_recv()
            pltpu.make_async_copy(
                x_ref.at[0, pl.ds(row0 + c * m_c, m_c), pl.ds(keep_off, n_half)],
                va.at[s],
                la.at[s],
            ).start()
            pltpu.make_async_copy(
                zrecv_ref.at[pl.ds(c * m_c, m_c), :], vb.at[s], lb.at[s]
            ).start()
            pltpu.make_async_copy(
                x_ref.at[0, pl.ds(row0 + c * m_c, m_c), pl.ds(keep_off, n_half)],
                va.at[s],
                la.at[s],
            ).wait()
            pltpu.make_async_copy(
                zrecv_ref.at[pl.ds(c * m_c, m_c), :], vb.at[s], lb.at[s]
            ).wait()
            vs[s] = va[s] + vb[s]

            pltpu.make_async_copy(
                vs.at[s], out_ref.at[pl.ds(row0 + c * m_c, m_c), :], lo.at[s]
            ).start()
            bcast(c, s, cw, sf_cw, rf_left).start()
            bcast(c, s, ccw, sf_ccw, rf_right).start()

            full_wait(c, left_q, rf_left).wait_recv()
            relay(c, left_q, 0, cw, sh_cw, rh_left).start()
            full_wait(c, right_q, rf_right).wait_recv()
            relay(c, right_q, 1, ccw, sh_ccw, rh_right).start()

        for c in range(N_CHUNKS - 2, N_CHUNKS):
            s = c % 2
            pltpu.make_async_copy(
                vs.at[s], out_ref.at[pl.ds(row0 + c * m_c, m_c), :], lo.at[s]
            ).wait()
            bcast(c, s, cw, sf_cw, rf_left).wait_send()
            bcast(c, s, ccw, sf_ccw, rf_right).wait_send()
        for c in range(N_CHUNKS):
            z_rdma(c).wait_send()
        for c in range(N_CHUNKS):
            relay(c, diag_q, 0, cw, sh_cw, rh_left).wait_recv()
            relay(c, diag_q, 1, ccw, sh_ccw, rh_right).wait_recv()
        for c in range(N_CHUNKS):
            relay(c, left_q, 0, cw, sh_cw, rh_left).wait_send()
            relay(c, right_q, 1, ccw, sh_ccw, rh_right).wait_send()

    out, _ = pl.pallas_call(
        body,
        out_shape=(
            jax.ShapeDtypeStruct((m_total, n_half), jnp.float32),
            jax.ShapeDtypeStruct((m_q, n_half), jnp.float32),
        ),
        in_specs=[pl.BlockSpec(memory_space=pl.ANY)],
        out_specs=(
            pl.BlockSpec(memory_space=pl.ANY),
            pl.BlockSpec(memory_space=pl.ANY),
        ),
        scratch_shapes=[
            pltpu.SemaphoreType.DMA((N_CHUNKS,)),
            pltpu.SemaphoreType.DMA((N_CHUNKS,)),
            pltpu.SemaphoreType.DMA((N_CHUNKS,)),
            pltpu.SemaphoreType.DMA((N_CHUNKS,)),
            pltpu.SemaphoreType.DMA((N_CHUNKS,)),
            pltpu.SemaphoreType.DMA((N_CHUNKS,)),
            pltpu.SemaphoreType.DMA((N_CHUNKS,)),
            pltpu.SemaphoreType.DMA((N_CHUNKS,)),
            pltpu.SemaphoreType.DMA((N_CHUNKS,)),
            pltpu.SemaphoreType.DMA((N_CHUNKS,)),
            pltpu.SemaphoreType.DMA((2,)),
            pltpu.SemaphoreType.DMA((2,)),
            pltpu.SemaphoreType.DMA((2,)),
            pltpu.VMEM((2, m_c, n_half), jnp.float32),
            pltpu.VMEM((2, m_c, n_half), jnp.float32),
            pltpu.VMEM((2, m_c, n_half), jnp.float32),
        ],
        compiler_params=pltpu.CompilerParams(collective_id=0),
    )(x)
    return out
